# baseline (speedup 1.0000x reference)
import functools

import numpy as np

import jax
import jax.numpy as jnp
from jax import lax
from jax.experimental import pallas as pl
from jax.experimental.pallas import tpu as pltpu
from jax.experimental.pallas import tpu_sc as plsc

N_D = 25000
N_S = 25000
E = 800000
Q = 100000
H = 64

NS = 16
CHUNK = 128
K = 2
KC = 8

E_ROWS = 6272
E_PAD = E_ROWS * CHUNK
TILE_ROWS = E_ROWS // NS
N_ACC = 25088
Z_ROWS = N_ACC // NS
CW = 16

P_ROWS = 1568
P_PAD = P_ROWS * CHUNK
P_TILE = P_ROWS // NS
KP = 7


def _proj2_kernel(xp_ref, wp_ref, bp_ref, ep_ref, xs_ref, ws_ref, bs_ref,
                  es_ref, op_ref, os_ref):
    op_ref[...] = (
        jnp.dot(xp_ref[...], wp_ref[...], preferred_element_type=jnp.float32)
        + bp_ref[...]
        + ep_ref[...]
    )
    os_ref[...] = (
        jnp.dot(xs_ref[...], ws_ref[...], preferred_element_type=jnp.float32)
        + bs_ref[...]
        + es_ref[...]
    )


def _proj2(xp, wp, bp, ep, xs, ws, bs, es, block_rows=1000):
    n, fp = xp.shape
    fs = xs.shape[1]
    return pl.pallas_call(
        _proj2_kernel,
        grid=(n // block_rows,),
        in_specs=[
            pl.BlockSpec((block_rows, fp), lambda i: (i, 0)),
            pl.BlockSpec((fp, H), lambda i: (0, 0)),
            pl.BlockSpec((1, H), lambda i: (0, 0)),
            pl.BlockSpec((block_rows, H), lambda i: (i, 0)),
            pl.BlockSpec((block_rows, fs), lambda i: (i, 0)),
            pl.BlockSpec((fs, H), lambda i: (0, 0)),
            pl.BlockSpec((1, H), lambda i: (0, 0)),
            pl.BlockSpec((block_rows, H), lambda i: (i, 0)),
        ],
        out_specs=[
            pl.BlockSpec((block_rows, H), lambda i: (i, 0)),
            pl.BlockSpec((block_rows, H), lambda i: (i, 0)),
        ],
        out_shape=[jax.ShapeDtypeStruct((n, H), jnp.float32)] * 2,
    )(xp, wp, bp.reshape(1, H), ep, xs, ws, bs.reshape(1, H), es)


def _combine_kernel(ss_ref, cs_ref, hs_ref, wls_ref, bs_ref, wrs_ref,
                    sp_ref, cp_ref, hp_ref, wlp_ref, bp_ref, wrp_ref,
                    os_ref, op_ref, *, relu):
    def one(s_ref, c_ref, h_ref, wl_ref, b_ref, wr_ref, o_ref):
        cnt = jnp.maximum(c_ref[:, :1], 1.0)
        m = s_ref[...] / cnt
        acc = (
            jnp.dot(m, wl_ref[...], preferred_element_type=jnp.float32)
            + b_ref[...]
            + jnp.dot(h_ref[...], wr_ref[...],
                      preferred_element_type=jnp.float32)
        )
        o_ref[...] = jnp.maximum(acc, 0.0) if relu else acc
    one(ss_ref, cs_ref, hs_ref, wls_ref, bs_ref, wrs_ref, os_ref)
    one(sp_ref, cp_ref, hp_ref, wlp_ref, bp_ref, wrp_ref, op_ref)


def _combine(ss, cs, hs, wls, bls, wrs, sp, cp, hp, wlp, blp, wrp, relu,
             block_rows=1000):
    nd = pl.BlockSpec((block_rows, H), lambda i: (i, 0))
    cd = pl.BlockSpec((block_rows, CW), lambda i: (i, 0))
    wd = pl.BlockSpec((H, H), lambda i: (0, 0))
    bd = pl.BlockSpec((1, H), lambda i: (0, 0))
    return pl.pallas_call(
        functools.partial(_combine_kernel, relu=relu),
        grid=(N_S // block_rows,),
        in_specs=[nd, cd, nd, wd, bd, wd, nd, cd, nd, wd, bd, wd],
        out_specs=[nd, nd],
        out_shape=[jax.ShapeDtypeStruct((N_S, H), jnp.float32)] * 2,
    )(ss, cs, hs, wls, bls.reshape(1, H), wrs,
      sp, cp, hp, wlp, blp.reshape(1, H), wrp)


def _dot_kernel(a_ref, b_ref, o_ref):
    o_ref[...] = jnp.sum(a_ref[...] * b_ref[...], axis=-1).reshape(1, 1, -1)


def _pair_dot(a, b, n, block_rows=1000):
    nblk = n // block_rows
    out = pl.pallas_call(
        _dot_kernel,
        grid=(nblk,),
        in_specs=[
            pl.BlockSpec((block_rows, H), lambda i: (i, 0)),
            pl.BlockSpec((block_rows, H), lambda i: (i, 0)),
        ],
        out_specs=pl.BlockSpec((1, 1, block_rows), lambda i: (i, 0, 0)),
        out_shape=jax.ShapeDtypeStruct((nblk, 1, block_rows), jnp.float32),
    )(a, b)
    return out.reshape(n)


def _sc_params():
    return pltpu.CompilerParams(use_tc_tiling_on_sc=False)


@functools.lru_cache(maxsize=None)
def _make_aggr():
    mesh = plsc.VectorSubcoreMesh(core_axis_name="c", subcore_axis_name="s")

    @functools.partial(
        pl.kernel,
        out_type=[jax.ShapeDtypeStruct((N_ACC, H), jnp.float32)] * 2,
        mesh=mesh,
        compiler_params=_sc_params(),
        scratch_types=[
            pltpu.VMEM((2, 2, CHUNK), jnp.int32),
            pltpu.VMEM((2, 2, CHUNK), jnp.int32),
            pltpu.VMEM((CHUNK, H), jnp.float32),
            pltpu.VMEM((CHUNK, H), jnp.float32),
            pltpu.VMEM_SHARED((N_ACC, H), jnp.float32),
            pltpu.SemaphoreType.DMA,
            pltpu.SemaphoreType.DMA,
            pltpu.SemaphoreType.DMA,
            pltpu.SemaphoreType.DMA,
            pltpu.SemaphoreType.DMA,
        ],
    )
    def aggr(ta, tb, ia, ib, zz, oa, ob, idx0, idx1, rowsa, rowsb, acc,
             semg, sema, semb, semi0, semi1):
        c = lax.axis_index("c")
        s = lax.axis_index("s")
        pltpu.sync_copy(zz, acc.at[pl.ds(s * Z_ROWS, Z_ROWS)])
        plsc.subcore_barrier()

        def run_dir(table, idx_hbm):
            c0 = s * TILE_ROWS

            def drain_scatters():
                pltpu.make_async_copy(
                    table.at[pl.ds(0, CHUNK)], rowsa, sema).wait()
                pltpu.make_async_copy(
                    table.at[pl.ds(0, CHUNK)], rowsb, semb).wait()

            def step(idx, semi_cur, row_base, pre_row, pre_buf, pre_sem,
                     drain_sc, drain_i):
                if drain_sc:
                    pltpu.make_async_copy(
                        table.at[pl.ds(0, CHUNK)], rowsa, sema).wait()
                if pre_buf is not None:
                    pltpu.async_copy(
                        idx_hbm.at[pl.ds(pre_row, 2)], pre_buf, pre_sem)
                if drain_i:
                    pltpu.make_async_copy(
                        idx_hbm.at[pl.ds(0, 2)], idx, semi_cur).wait()
                cpa = pltpu.async_copy(table.at[idx.at[0, 0]], rowsa, semg)
                cpa.wait()
                pltpu.async_copy(rowsa, acc.at[idx.at[0, 1]], sema, add=True)
                if drain_sc:
                    pltpu.make_async_copy(
                        table.at[pl.ds(0, CHUNK)], rowsb, semb).wait()
                cpb = pltpu.async_copy(table.at[idx.at[1, 0]], rowsb, semg)
                cpb.wait()
                pltpu.async_copy(rowsb, acc.at[idx.at[1, 1]], semb, add=True)

            pltpu.sync_copy(idx_hbm.at[pl.ds(c0, 2)], idx0)
            step(idx0, semi0, c0, c0 + 2, idx1, semi1,
                 drain_sc=False, drain_i=False)

            def body(k, carry):
                base = c0 + 2 * (2 * k + 1)
                step(idx1, semi1, base, base + 2, idx0, semi0,
                     drain_sc=True, drain_i=True)
                step(idx0, semi0, base + 2, base + 4, idx1, semi1,
                     drain_sc=True, drain_i=True)
                return carry
            lax.fori_loop(0, TILE_ROWS // 4 - 1, body, 0)

            step(idx1, semi1, c0 + TILE_ROWS - 2, None, None, None,
                 drain_sc=True, drain_i=True)
            drain_scatters()

        @pl.when(c == 0)
        def _():
            run_dir(ta, ia)

        @pl.when(c == 1)
        def _():
            run_dir(tb, ib)

        plsc.subcore_barrier()

        @pl.when(c == 0)
        def _():
            pltpu.sync_copy(acc.at[pl.ds(s * Z_ROWS, Z_ROWS)],
                            oa.at[pl.ds(s * Z_ROWS, Z_ROWS)])

        @pl.when(c == 1)
        def _():
            pltpu.sync_copy(acc.at[pl.ds(s * Z_ROWS, Z_ROWS)],
                            ob.at[pl.ds(s * Z_ROWS, Z_ROWS)])

    return aggr


@functools.lru_cache(maxsize=None)
def _make_counts():
    mesh = plsc.VectorSubcoreMesh(core_axis_name="c", subcore_axis_name="s")

    @functools.partial(
        pl.kernel,
        out_type=[jax.ShapeDtypeStruct((N_ACC, CW), jnp.float32)] * 2,
        mesh=mesh,
        compiler_params=_sc_params(),
        scratch_types=[
            pltpu.VMEM((KC, CHUNK), jnp.int32),
            pltpu.VMEM((CHUNK, CW), jnp.float32),
            pltpu.VMEM_SHARED((N_ACC, CW), jnp.float32),
            pltpu.SemaphoreType.DMA,
        ],
    )
    def counts(sa, sb, ones, zz, oa, ob, sidx, ones_v, acc, sem):
        c = lax.axis_index("c")
        s = lax.axis_index("s")
        pltpu.sync_copy(ones, ones_v)
        pltpu.sync_copy(zz, acc.at[pl.ds(s * Z_ROWS, Z_ROWS)])
        plsc.subcore_barrier()

        def run_dir(sidx_hbm):
            def body(g, carry):
                row0 = s * TILE_ROWS + g * KC
                pltpu.sync_copy(sidx_hbm.at[pl.ds(row0, KC)], sidx)
                for j in range(KC):
                    pltpu.sync_copy(ones_v, acc.at[sidx.at[j]], add=True)
                return carry
            lax.fori_loop(0, TILE_ROWS // KC, body, 0)

        @pl.when(c == 0)
        def _():
            run_dir(sa)

        @pl.when(c == 1)
        def _():
            run_dir(sb)

        plsc.subcore_barrier()

        @pl.when(c == 0)
        def _():
            pltpu.sync_copy(acc.at[pl.ds(s * Z_ROWS, Z_ROWS)],
                            oa.at[pl.ds(s * Z_ROWS, Z_ROWS)])

        @pl.when(c == 1)
        def _():
            pltpu.sync_copy(acc.at[pl.ds(s * Z_ROWS, Z_ROWS)],
                            ob.at[pl.ds(s * Z_ROWS, Z_ROWS)])

    return counts


@functools.lru_cache(maxsize=None)
def _make_cls_gather():
    mesh = plsc.VectorSubcoreMesh(core_axis_name="c", subcore_axis_name="s")

    @functools.partial(
        pl.kernel,
        out_type=[jax.ShapeDtypeStruct((P_PAD, H), jnp.float32)] * 2,
        mesh=mesh,
        compiler_params=_sc_params(),
        scratch_types=[
            pltpu.VMEM((KP, CHUNK), jnp.int32),
            pltpu.VMEM((KP, CHUNK, H), jnp.float32),
            pltpu.SemaphoreType.DMA,
        ],
    )
    def cls_gather(ta, tb, aidx, bidx, oa, ob, idx, rows, sem):
        c = lax.axis_index("c")
        s = lax.axis_index("s")

        def run(table, idx_hbm, out):
            def body(g, carry):
                row0 = s * P_TILE + g * KP
                pltpu.sync_copy(idx_hbm.at[pl.ds(row0, KP)], idx)
                cps = [
                    pltpu.async_copy(table.at[idx.at[j]], rows.at[j], sem)
                    for j in range(KP)
                ]
                for cp in cps:
                    cp.wait()
                for j in range(KP):
                    pltpu.sync_copy(
                        rows.at[j], out.at[pl.ds((row0 + j) * CHUNK, CHUNK)])
                return carry
            lax.fori_loop(0, P_TILE // KP, body, 0)

        @pl.when(c == 0)
        def _():
            run(ta, aidx, oa)

        @pl.when(c == 1)
        def _():
            run(tb, bidx, ob)

    return cls_gather


def _neg_tails():
    k1, k2 = jax.random.split(jax.random.key(42))
    qpad = P_PAD - 2 * Q
    na = np.asarray(jax.random.randint(k1, (Q,), 0, N_D, dtype=jnp.int32))
    nb = np.asarray(jax.random.randint(k2, (Q,), 0, N_S, dtype=jnp.int32))
    zpad = np.zeros((qpad,), np.int32)
    return (np.concatenate([na, zpad]), np.concatenate([nb, zpad]))


_NEG_TAILS = _neg_tails()

def kernel(x_pdrugs, x_seffect, node_id_pdrugs, node_id_seffect, edge_index,
           edge_label_index, edge_label, params):
    src = edge_index[0]
    dst = edge_index[1]
    npad = E_PAD - E
    ga = jnp.concatenate([src, jnp.zeros((npad,), jnp.int32)]).reshape(E_ROWS, CHUNK)
    sa = jnp.concatenate([dst, jnp.full((npad,), N_S, jnp.int32)]).reshape(E_ROWS, CHUNK)
    gb = jnp.concatenate([dst, jnp.zeros((npad,), jnp.int32)]).reshape(E_ROWS, CHUNK)
    sb = jnp.concatenate([src, jnp.full((npad,), N_D, jnp.int32)]).reshape(E_ROWS, CHUNK)
    ia = jnp.stack([ga, sa], axis=1)
    ib = jnp.stack([gb, sb], axis=1)

    h_pd, h_se = _proj2(x_pdrugs, params["W_pd"], params["b_pd"],
                        params["emb_pd"], x_seffect, params["W_se"],
                        params["b_se"], params["emb_se"])

    ones = jnp.zeros((CHUNK, CW), jnp.float32).at[:, 0].set(1.0)
    zz16 = jnp.zeros((Z_ROWS, CW), jnp.float32)
    cnt_se, cnt_pd = _make_counts()(sa, sb, ones, zz16)

    z64 = jnp.zeros((Z_ROWS, H), jnp.float32)
    sum_se, sum_pd = _make_aggr()(h_pd, h_se, ia, ib, z64)

    h_se1, h_pd1 = _combine(
        sum_se, cnt_se, h_se, params["W1_ps_l"], params["b1_ps"],
        params["W1_ps_r"], sum_pd, cnt_pd, h_pd, params["W1_sp_l"],
        params["b1_sp"], params["W1_sp_r"], relu=True)

    sum_se2, sum_pd2 = _make_aggr()(h_pd1, h_se1, ia, ib, z64)

    h_se2, h_pd2 = _combine(
        sum_se2, cnt_se, h_se1, params["W2_ps_l"], params["b2_ps"],
        params["W2_ps_r"], sum_pd2, cnt_pd, h_pd1, params["W2_sp_l"],
        params["b2_sp"], params["W2_sp_r"], relu=False)

    na_tail, nb_tail = _NEG_TAILS
    aidx = jnp.concatenate([edge_label_index[0],
                            jnp.asarray(na_tail)]).reshape(P_ROWS, CHUNK)
    bidx = jnp.concatenate([edge_label_index[1],
                            jnp.asarray(nb_tail)]).reshape(P_ROWS, CHUNK)
    rows_a, rows_b = _make_cls_gather()(h_pd2, h_se2, aidx, bidx)
    pred = _pair_dot(rows_a, rows_b, 2 * Q)
    el = jnp.concatenate([edge_label, jnp.zeros((Q,), jnp.float32)])
    return pred, el


# speedup vs baseline: 1.0412x; 1.0412x over previous
import functools

import numpy as np

import jax
import jax.numpy as jnp
from jax import lax
from jax.experimental import pallas as pl
from jax.experimental.pallas import tpu as pltpu
from jax.experimental.pallas import tpu_sc as plsc

N_D = 25000
N_S = 25000
E = 800000
Q = 100000
H = 64

NS = 16
CHUNK = 128
K = 2
KC = 8

E_ROWS = 6272
E_PAD = E_ROWS * CHUNK
TILE_ROWS = E_ROWS // NS
N_ACC = 25088
Z_ROWS = N_ACC // NS
CW = 16

P_ROWS = 1568
P_PAD = P_ROWS * CHUNK
P_TILE = P_ROWS // NS
KP = 7


def _proj2_kernel(xp_ref, wp_ref, bp_ref, ep_ref, xs_ref, ws_ref, bs_ref,
                  es_ref, op_ref, os_ref):
    op_ref[...] = (
        jnp.dot(xp_ref[...], wp_ref[...], preferred_element_type=jnp.float32)
        + bp_ref[...]
        + ep_ref[...]
    )
    os_ref[...] = (
        jnp.dot(xs_ref[...], ws_ref[...], preferred_element_type=jnp.float32)
        + bs_ref[...]
        + es_ref[...]
    )


def _proj2(xp, wp, bp, ep, xs, ws, bs, es, block_rows=1000):
    n, fp = xp.shape
    fs = xs.shape[1]
    return pl.pallas_call(
        _proj2_kernel,
        grid=(n // block_rows,),
        in_specs=[
            pl.BlockSpec((block_rows, fp), lambda i: (i, 0)),
            pl.BlockSpec((fp, H), lambda i: (0, 0)),
            pl.BlockSpec((1, H), lambda i: (0, 0)),
            pl.BlockSpec((block_rows, H), lambda i: (i, 0)),
            pl.BlockSpec((block_rows, fs), lambda i: (i, 0)),
            pl.BlockSpec((fs, H), lambda i: (0, 0)),
            pl.BlockSpec((1, H), lambda i: (0, 0)),
            pl.BlockSpec((block_rows, H), lambda i: (i, 0)),
        ],
        out_specs=[
            pl.BlockSpec((block_rows, H), lambda i: (i, 0)),
            pl.BlockSpec((block_rows, H), lambda i: (i, 0)),
        ],
        out_shape=[jax.ShapeDtypeStruct((n, H), jnp.float32)] * 2,
    )(xp, wp, bp.reshape(1, H), ep, xs, ws, bs.reshape(1, H), es)


def _combine_kernel(ss_ref, cs_ref, hs_ref, wls_ref, bs_ref, wrs_ref,
                    sp_ref, cp_ref, hp_ref, wlp_ref, bp_ref, wrp_ref,
                    os_ref, op_ref, *, relu):
    def one(s_ref, c_ref, h_ref, wl_ref, b_ref, wr_ref, o_ref):
        cnt = jnp.maximum(c_ref[:, :1], 1.0)
        m = s_ref[...] / cnt
        acc = (
            jnp.dot(m, wl_ref[...], preferred_element_type=jnp.float32)
            + b_ref[...]
            + jnp.dot(h_ref[...], wr_ref[...],
                      preferred_element_type=jnp.float32)
        )
        o_ref[...] = jnp.maximum(acc, 0.0) if relu else acc
    one(ss_ref, cs_ref, hs_ref, wls_ref, bs_ref, wrs_ref, os_ref)
    one(sp_ref, cp_ref, hp_ref, wlp_ref, bp_ref, wrp_ref, op_ref)


def _combine(ss, cs, hs, wls, bls, wrs, sp, cp, hp, wlp, blp, wrp, relu,
             block_rows=1000):
    nd = pl.BlockSpec((block_rows, H), lambda i: (i, 0))
    cd = pl.BlockSpec((block_rows, CW), lambda i: (i, 0))
    wd = pl.BlockSpec((H, H), lambda i: (0, 0))
    bd = pl.BlockSpec((1, H), lambda i: (0, 0))
    return pl.pallas_call(
        functools.partial(_combine_kernel, relu=relu),
        grid=(N_S // block_rows,),
        in_specs=[nd, cd, nd, wd, bd, wd, nd, cd, nd, wd, bd, wd],
        out_specs=[nd, nd],
        out_shape=[jax.ShapeDtypeStruct((N_S, H), jnp.float32)] * 2,
    )(ss, cs, hs, wls, bls.reshape(1, H), wrs,
      sp, cp, hp, wlp, blp.reshape(1, H), wrp)


def _dot_kernel(a_ref, b_ref, o_ref):
    o_ref[...] = jnp.sum(a_ref[...] * b_ref[...], axis=-1).reshape(1, 1, -1)


def _pair_dot(a, b, n, block_rows=1000):
    nblk = n // block_rows
    out = pl.pallas_call(
        _dot_kernel,
        grid=(nblk,),
        in_specs=[
            pl.BlockSpec((block_rows, H), lambda i: (i, 0)),
            pl.BlockSpec((block_rows, H), lambda i: (i, 0)),
        ],
        out_specs=pl.BlockSpec((1, 1, block_rows), lambda i: (i, 0, 0)),
        out_shape=jax.ShapeDtypeStruct((nblk, 1, block_rows), jnp.float32),
    )(a, b)
    return out.reshape(n)


def _sc_params():
    return pltpu.CompilerParams(use_tc_tiling_on_sc=False)


@functools.lru_cache(maxsize=None)
def _make_aggr():
    mesh = plsc.VectorSubcoreMesh(core_axis_name="c", subcore_axis_name="s")

    @functools.partial(
        pl.kernel,
        out_type=[jax.ShapeDtypeStruct((N_ACC, H), jnp.float32)] * 2,
        mesh=mesh,
        compiler_params=_sc_params(),
        scratch_types=[
            pltpu.VMEM((2, 2, CHUNK), jnp.int32),
            pltpu.VMEM((2, 2, CHUNK), jnp.int32),
            pltpu.VMEM((CHUNK, H), jnp.float32),
            pltpu.VMEM((CHUNK, H), jnp.float32),
            pltpu.VMEM_SHARED((N_ACC, H), jnp.float32),
            pltpu.SemaphoreType.DMA,
            pltpu.SemaphoreType.DMA,
            pltpu.SemaphoreType.DMA,
            pltpu.SemaphoreType.DMA,
            pltpu.SemaphoreType.DMA,
        ],
    )
    def aggr(ta, tb, ia, ib, zz, oa, ob, idx0, idx1, rowsa, rowsb, acc,
             semg, sema, semb, semi0, semi1):
        c = lax.axis_index("c")
        s = lax.axis_index("s")
        pltpu.sync_copy(zz, acc.at[pl.ds(s * Z_ROWS, Z_ROWS)])
        plsc.subcore_barrier()

        def run_dir(table, idx_hbm):
            c0 = s * TILE_ROWS

            def drain_scatters():
                pltpu.make_async_copy(
                    table.at[pl.ds(0, CHUNK)], rowsa, sema).wait()
                pltpu.make_async_copy(
                    table.at[pl.ds(0, CHUNK)], rowsb, semb).wait()

            def step(idx, semi_cur, row_base, pre_row, pre_buf, pre_sem,
                     drain_sc, drain_i):
                if drain_sc:
                    drain_scatters()
                if pre_buf is not None:
                    pltpu.async_copy(
                        idx_hbm.at[pl.ds(pre_row, 2)], pre_buf, pre_sem)
                if drain_i:
                    pltpu.make_async_copy(
                        idx_hbm.at[pl.ds(0, 2)], idx, semi_cur).wait()
                cpa = pltpu.async_copy(table.at[idx.at[0, 0]], rowsa, semg)
                cpb = pltpu.async_copy(table.at[idx.at[1, 0]], rowsb, semg)
                cpa.wait()
                pltpu.async_copy(rowsa, acc.at[idx.at[0, 1]], sema, add=True)
                cpb.wait()
                pltpu.async_copy(rowsb, acc.at[idx.at[1, 1]], semb, add=True)

            pltpu.sync_copy(idx_hbm.at[pl.ds(c0, 2)], idx0)
            step(idx0, semi0, c0, c0 + 2, idx1, semi1,
                 drain_sc=False, drain_i=False)

            def body(k, carry):
                base = c0 + 2 * (2 * k + 1)
                step(idx1, semi1, base, base + 2, idx0, semi0,
                     drain_sc=True, drain_i=True)
                step(idx0, semi0, base + 2, base + 4, idx1, semi1,
                     drain_sc=True, drain_i=True)
                return carry
            lax.fori_loop(0, TILE_ROWS // 4 - 1, body, 0)

            step(idx1, semi1, c0 + TILE_ROWS - 2, None, None, None,
                 drain_sc=True, drain_i=True)
            drain_scatters()

        @pl.when(c == 0)
        def _():
            run_dir(ta, ia)

        @pl.when(c == 1)
        def _():
            run_dir(tb, ib)

        plsc.subcore_barrier()

        @pl.when(c == 0)
        def _():
            pltpu.sync_copy(acc.at[pl.ds(s * Z_ROWS, Z_ROWS)],
                            oa.at[pl.ds(s * Z_ROWS, Z_ROWS)])

        @pl.when(c == 1)
        def _():
            pltpu.sync_copy(acc.at[pl.ds(s * Z_ROWS, Z_ROWS)],
                            ob.at[pl.ds(s * Z_ROWS, Z_ROWS)])

    return aggr


@functools.lru_cache(maxsize=None)
def _make_counts():
    mesh = plsc.VectorSubcoreMesh(core_axis_name="c", subcore_axis_name="s")

    @functools.partial(
        pl.kernel,
        out_type=[jax.ShapeDtypeStruct((N_ACC, CW), jnp.float32)] * 2,
        mesh=mesh,
        compiler_params=_sc_params(),
        scratch_types=[
            pltpu.VMEM((KC, CHUNK), jnp.int32),
            pltpu.VMEM((CHUNK, CW), jnp.float32),
            pltpu.VMEM_SHARED((N_ACC, CW), jnp.float32),
            pltpu.SemaphoreType.DMA,
        ],
    )
    def counts(sa, sb, ones, zz, oa, ob, sidx, ones_v, acc, sem):
        c = lax.axis_index("c")
        s = lax.axis_index("s")
        pltpu.sync_copy(ones, ones_v)
        pltpu.sync_copy(zz, acc.at[pl.ds(s * Z_ROWS, Z_ROWS)])
        plsc.subcore_barrier()

        def run_dir(sidx_hbm):
            def body(g, carry):
                row0 = s * TILE_ROWS + g * KC
                pltpu.sync_copy(sidx_hbm.at[pl.ds(row0, KC)], sidx)
                for j in range(KC):
                    pltpu.sync_copy(ones_v, acc.at[sidx.at[j]], add=True)
                return carry
            lax.fori_loop(0, TILE_ROWS // KC, body, 0)

        @pl.when(c == 0)
        def _():
            run_dir(sa)

        @pl.when(c == 1)
        def _():
            run_dir(sb)

        plsc.subcore_barrier()

        @pl.when(c == 0)
        def _():
            pltpu.sync_copy(acc.at[pl.ds(s * Z_ROWS, Z_ROWS)],
                            oa.at[pl.ds(s * Z_ROWS, Z_ROWS)])

        @pl.when(c == 1)
        def _():
            pltpu.sync_copy(acc.at[pl.ds(s * Z_ROWS, Z_ROWS)],
                            ob.at[pl.ds(s * Z_ROWS, Z_ROWS)])

    return counts


@functools.lru_cache(maxsize=None)
def _make_cls_gather():
    mesh = plsc.VectorSubcoreMesh(core_axis_name="c", subcore_axis_name="s")

    @functools.partial(
        pl.kernel,
        out_type=[jax.ShapeDtypeStruct((P_PAD, H), jnp.float32)] * 2,
        mesh=mesh,
        compiler_params=_sc_params(),
        scratch_types=[
            pltpu.VMEM((KP, CHUNK), jnp.int32),
            pltpu.VMEM((KP, CHUNK, H), jnp.float32),
            pltpu.SemaphoreType.DMA,
        ],
    )
    def cls_gather(ta, tb, aidx, bidx, oa, ob, idx, rows, sem):
        c = lax.axis_index("c")
        s = lax.axis_index("s")

        def run(table, idx_hbm, out):
            def body(g, carry):
                row0 = s * P_TILE + g * KP
                pltpu.sync_copy(idx_hbm.at[pl.ds(row0, KP)], idx)
                cps = [
                    pltpu.async_copy(table.at[idx.at[j]], rows.at[j], sem)
                    for j in range(KP)
                ]
                for cp in cps:
                    cp.wait()
                for j in range(KP):
                    pltpu.sync_copy(
                        rows.at[j], out.at[pl.ds((row0 + j) * CHUNK, CHUNK)])
                return carry
            lax.fori_loop(0, P_TILE // KP, body, 0)

        @pl.when(c == 0)
        def _():
            run(ta, aidx, oa)

        @pl.when(c == 1)
        def _():
            run(tb, bidx, ob)

    return cls_gather


@functools.lru_cache(maxsize=None)
def _neg_tails():
    with jax.ensure_compile_time_eval():
        k1, k2 = jax.random.split(jax.random.key(42))
        na = np.asarray(jax.random.randint(k1, (Q,), 0, N_D, dtype=jnp.int32))
        nb = np.asarray(jax.random.randint(k2, (Q,), 0, N_S, dtype=jnp.int32))
    qpad = P_PAD - 2 * Q
    zpad = np.zeros((qpad,), np.int32)
    return (np.concatenate([na, zpad]), np.concatenate([nb, zpad]))

def kernel(x_pdrugs, x_seffect, node_id_pdrugs, node_id_seffect, edge_index,
           edge_label_index, edge_label, params):
    src = edge_index[0]
    dst = edge_index[1]
    npad = E_PAD - E
    ga = jnp.concatenate([src, jnp.zeros((npad,), jnp.int32)]).reshape(E_ROWS, CHUNK)
    sa = jnp.concatenate([dst, jnp.full((npad,), N_S, jnp.int32)]).reshape(E_ROWS, CHUNK)
    gb = jnp.concatenate([dst, jnp.zeros((npad,), jnp.int32)]).reshape(E_ROWS, CHUNK)
    sb = jnp.concatenate([src, jnp.full((npad,), N_D, jnp.int32)]).reshape(E_ROWS, CHUNK)
    ia = jnp.stack([ga, sa], axis=1)
    ib = jnp.stack([gb, sb], axis=1)

    h_pd, h_se = _proj2(x_pdrugs, params["W_pd"], params["b_pd"],
                        params["emb_pd"], x_seffect, params["W_se"],
                        params["b_se"], params["emb_se"])

    ones = jnp.zeros((CHUNK, CW), jnp.float32).at[:, 0].set(1.0)
    zz16 = jnp.zeros((Z_ROWS, CW), jnp.float32)
    cnt_se, cnt_pd = _make_counts()(sa, sb, ones, zz16)

    z64 = jnp.zeros((Z_ROWS, H), jnp.float32)
    sum_se, sum_pd = _make_aggr()(h_pd, h_se, ia, ib, z64)

    h_se1, h_pd1 = _combine(
        sum_se, cnt_se, h_se, params["W1_ps_l"], params["b1_ps"],
        params["W1_ps_r"], sum_pd, cnt_pd, h_pd, params["W1_sp_l"],
        params["b1_sp"], params["W1_sp_r"], relu=True)

    sum_se2, sum_pd2 = _make_aggr()(h_pd1, h_se1, ia, ib, z64)

    h_se2, h_pd2 = _combine(
        sum_se2, cnt_se, h_se1, params["W2_ps_l"], params["b2_ps"],
        params["W2_ps_r"], sum_pd2, cnt_pd, h_pd1, params["W2_sp_l"],
        params["b2_sp"], params["W2_sp_r"], relu=False)

    na_tail, nb_tail = _neg_tails()
    aidx = jnp.concatenate([edge_label_index[0],
                            jnp.asarray(na_tail)]).reshape(P_ROWS, CHUNK)
    bidx = jnp.concatenate([edge_label_index[1],
                            jnp.asarray(nb_tail)]).reshape(P_ROWS, CHUNK)
    rows_a, rows_b = _make_cls_gather()(h_pd2, h_se2, aidx, bidx)
    pred = _pair_dot(rows_a, rows_b, 2 * Q)
    el = jnp.concatenate([edge_label, jnp.zeros((Q,), jnp.float32)])
    return pred, el


# speedup vs baseline: 1.0693x; 1.0270x over previous
import functools

import numpy as np

import jax
import jax.numpy as jnp
from jax import lax
from jax.experimental import pallas as pl
from jax.experimental.pallas import tpu as pltpu
from jax.experimental.pallas import tpu_sc as plsc

N_D = 25000
N_S = 25000
E = 800000
Q = 100000
H = 64

NS = 16
CHUNK = 128
K = 2
KC = 7

E_ROWS = 6272
E_PAD = E_ROWS * CHUNK
TILE_ROWS = E_ROWS // NS
N_ACC = 25088
Z_ROWS = N_ACC // NS
CW = 16

P_ROWS = 1568
P_PAD = P_ROWS * CHUNK
P_TILE = P_ROWS // NS
KP = 7


def _proj2_kernel(xp_ref, wp_ref, bp_ref, ep_ref, xs_ref, ws_ref, bs_ref,
                  es_ref, op_ref, os_ref):
    op_ref[...] = (
        jnp.dot(xp_ref[...], wp_ref[...], preferred_element_type=jnp.float32)
        + bp_ref[...]
        + ep_ref[...]
    )
    os_ref[...] = (
        jnp.dot(xs_ref[...], ws_ref[...], preferred_element_type=jnp.float32)
        + bs_ref[...]
        + es_ref[...]
    )


def _proj2(xp, wp, bp, ep, xs, ws, bs, es, block_rows=1000):
    n, fp = xp.shape
    fs = xs.shape[1]
    return pl.pallas_call(
        _proj2_kernel,
        grid=(n // block_rows,),
        in_specs=[
            pl.BlockSpec((block_rows, fp), lambda i: (i, 0)),
            pl.BlockSpec((fp, H), lambda i: (0, 0)),
            pl.BlockSpec((1, H), lambda i: (0, 0)),
            pl.BlockSpec((block_rows, H), lambda i: (i, 0)),
            pl.BlockSpec((block_rows, fs), lambda i: (i, 0)),
            pl.BlockSpec((fs, H), lambda i: (0, 0)),
            pl.BlockSpec((1, H), lambda i: (0, 0)),
            pl.BlockSpec((block_rows, H), lambda i: (i, 0)),
        ],
        out_specs=[
            pl.BlockSpec((block_rows, H), lambda i: (i, 0)),
            pl.BlockSpec((block_rows, H), lambda i: (i, 0)),
        ],
        out_shape=[jax.ShapeDtypeStruct((n, H), jnp.float32)] * 2,
    )(xp, wp, bp.reshape(1, H), ep, xs, ws, bs.reshape(1, H), es)


def _combine_kernel(ss_ref, cs_ref, hs_ref, wls_ref, bs_ref, wrs_ref,
                    sp_ref, cp_ref, hp_ref, wlp_ref, bp_ref, wrp_ref,
                    os_ref, op_ref, *, relu):
    def one(s_ref, c_ref, h_ref, wl_ref, b_ref, wr_ref, o_ref):
        cnt = jnp.maximum(c_ref[:, :1], 1.0)
        m = s_ref[...] / cnt
        acc = (
            jnp.dot(m, wl_ref[...], preferred_element_type=jnp.float32)
            + b_ref[...]
            + jnp.dot(h_ref[...], wr_ref[...],
                      preferred_element_type=jnp.float32)
        )
        o_ref[...] = jnp.maximum(acc, 0.0) if relu else acc
    one(ss_ref, cs_ref, hs_ref, wls_ref, bs_ref, wrs_ref, os_ref)
    one(sp_ref, cp_ref, hp_ref, wlp_ref, bp_ref, wrp_ref, op_ref)


def _combine(ss, cs, hs, wls, bls, wrs, sp, cp, hp, wlp, blp, wrp, relu,
             block_rows=1000):
    nd = pl.BlockSpec((block_rows, H), lambda i: (i, 0))
    cd = pl.BlockSpec((block_rows, CW), lambda i: (i, 0))
    wd = pl.BlockSpec((H, H), lambda i: (0, 0))
    bd = pl.BlockSpec((1, H), lambda i: (0, 0))
    return pl.pallas_call(
        functools.partial(_combine_kernel, relu=relu),
        grid=(N_S // block_rows,),
        in_specs=[nd, cd, nd, wd, bd, wd, nd, cd, nd, wd, bd, wd],
        out_specs=[nd, nd],
        out_shape=[jax.ShapeDtypeStruct((N_S, H), jnp.float32)] * 2,
    )(ss, cs, hs, wls, bls.reshape(1, H), wrs,
      sp, cp, hp, wlp, blp.reshape(1, H), wrp)


def _dot_kernel(a_ref, b_ref, o_ref):
    o_ref[...] = jnp.sum(a_ref[...] * b_ref[...], axis=-1).reshape(1, 1, -1)


def _pair_dot(a, b, n, block_rows=1000):
    nblk = n // block_rows
    out = pl.pallas_call(
        _dot_kernel,
        grid=(nblk,),
        in_specs=[
            pl.BlockSpec((block_rows, H), lambda i: (i, 0)),
            pl.BlockSpec((block_rows, H), lambda i: (i, 0)),
        ],
        out_specs=pl.BlockSpec((1, 1, block_rows), lambda i: (i, 0, 0)),
        out_shape=jax.ShapeDtypeStruct((nblk, 1, block_rows), jnp.float32),
    )(a, b)
    return out.reshape(n)


def _sc_params():
    return pltpu.CompilerParams(use_tc_tiling_on_sc=False)


@functools.lru_cache(maxsize=None)
def _make_aggr():
    mesh = plsc.VectorSubcoreMesh(core_axis_name="c", subcore_axis_name="s")

    @functools.partial(
        pl.kernel,
        out_type=[jax.ShapeDtypeStruct((N_ACC, H), jnp.float32)] * 2,
        mesh=mesh,
        compiler_params=_sc_params(),
        scratch_types=[
            pltpu.VMEM((2, 2, CHUNK), jnp.int32),
            pltpu.VMEM((2, 2, CHUNK), jnp.int32),
            pltpu.VMEM((CHUNK, H), jnp.float32),
            pltpu.VMEM((CHUNK, H), jnp.float32),
            pltpu.VMEM_SHARED((N_ACC, H), jnp.float32),
            pltpu.SemaphoreType.DMA,
            pltpu.SemaphoreType.DMA,
            pltpu.SemaphoreType.DMA,
            pltpu.SemaphoreType.DMA,
            pltpu.SemaphoreType.DMA,
        ],
    )
    def aggr(ta, tb, ia, ib, zz, oa, ob, idx0, idx1, rowsa, rowsb, acc,
             semg, sema, semb, semi0, semi1):
        c = lax.axis_index("c")
        s = lax.axis_index("s")
        pltpu.sync_copy(zz, acc.at[pl.ds(s * Z_ROWS, Z_ROWS)])
        plsc.subcore_barrier()

        def run_dir(table, idx_hbm):
            c0 = s * TILE_ROWS

            def drain_scatters():
                pltpu.make_async_copy(
                    table.at[pl.ds(0, CHUNK)], rowsa, sema).wait()
                pltpu.make_async_copy(
                    table.at[pl.ds(0, CHUNK)], rowsb, semb).wait()

            def step(idx, semi_cur, row_base, pre_row, pre_buf, pre_sem,
                     drain_sc, drain_i):
                if drain_sc:
                    drain_scatters()
                if pre_buf is not None:
                    pltpu.async_copy(
                        idx_hbm.at[pl.ds(pre_row, 2)], pre_buf, pre_sem)
                if drain_i:
                    pltpu.make_async_copy(
                        idx_hbm.at[pl.ds(0, 2)], idx, semi_cur).wait()
                cpa = pltpu.async_copy(table.at[idx.at[0, 0]], rowsa, semg)
                cpb = pltpu.async_copy(table.at[idx.at[1, 0]], rowsb, semg)
                cpa.wait()
                pltpu.async_copy(rowsa, acc.at[idx.at[0, 1]], sema, add=True)
                cpb.wait()
                pltpu.async_copy(rowsb, acc.at[idx.at[1, 1]], semb, add=True)

            pltpu.sync_copy(idx_hbm.at[pl.ds(c0, 2)], idx0)
            step(idx0, semi0, c0, c0 + 2, idx1, semi1,
                 drain_sc=False, drain_i=False)

            def body(k, carry):
                base = c0 + 2 * (2 * k + 1)
                step(idx1, semi1, base, base + 2, idx0, semi0,
                     drain_sc=True, drain_i=True)
                step(idx0, semi0, base + 2, base + 4, idx1, semi1,
                     drain_sc=True, drain_i=True)
                return carry
            lax.fori_loop(0, TILE_ROWS // 4 - 1, body, 0)

            step(idx1, semi1, c0 + TILE_ROWS - 2, None, None, None,
                 drain_sc=True, drain_i=True)
            drain_scatters()

        @pl.when(c == 0)
        def _():
            run_dir(ta, ia)

        @pl.when(c == 1)
        def _():
            run_dir(tb, ib)

        plsc.subcore_barrier()

        @pl.when(c == 0)
        def _():
            pltpu.sync_copy(acc.at[pl.ds(s * Z_ROWS, Z_ROWS)],
                            oa.at[pl.ds(s * Z_ROWS, Z_ROWS)])

        @pl.when(c == 1)
        def _():
            pltpu.sync_copy(acc.at[pl.ds(s * Z_ROWS, Z_ROWS)],
                            ob.at[pl.ds(s * Z_ROWS, Z_ROWS)])

    return aggr


@functools.lru_cache(maxsize=None)
def _make_counts():
    mesh = plsc.VectorSubcoreMesh(core_axis_name="c", subcore_axis_name="s")

    @functools.partial(
        pl.kernel,
        out_type=[jax.ShapeDtypeStruct((N_ACC, CW), jnp.float32)] * 2,
        mesh=mesh,
        compiler_params=_sc_params(),
        scratch_types=[
            pltpu.VMEM((KC, CHUNK), jnp.int32),
            pltpu.VMEM((KC, CHUNK), jnp.int32),
            pltpu.VMEM((CHUNK, CW), jnp.float32),
            pltpu.VMEM_SHARED((N_ACC, CW), jnp.float32),
            pltpu.SemaphoreType.DMA,
            pltpu.SemaphoreType.DMA,
        ],
    )
    def counts(sa, sb, ones, zz, oa, ob, sidxa, sidxb, ones_v, acc,
               sema, semb):
        c = lax.axis_index("c")
        s = lax.axis_index("s")
        pltpu.sync_copy(ones, ones_v)
        pltpu.sync_copy(zz, acc.at[pl.ds(s * Z_ROWS, Z_ROWS)])
        plsc.subcore_barrier()

        def run_dir(sidx_hbm):
            def half(row0, sidx, sem, drain):
                if drain:
                    for _ in range(KC):
                        pltpu.make_async_copy(
                            ones, ones_v, sem).wait()
                pltpu.sync_copy(sidx_hbm.at[pl.ds(row0, KC)], sidx)
                for j in range(KC):
                    pltpu.async_copy(ones_v, acc.at[sidx.at[j]], sem,
                                     add=True)

            def body(g, carry):
                row0 = s * TILE_ROWS + g * 2 * KC
                half(row0, sidxa, sema, True)
                half(row0 + KC, sidxb, semb, True)
                return carry
            half(s * TILE_ROWS, sidxa, sema, False)
            half(s * TILE_ROWS + KC, sidxb, semb, False)
            lax.fori_loop(1, TILE_ROWS // (2 * KC), body, 0)
            for _ in range(KC):
                pltpu.make_async_copy(ones, ones_v, sema).wait()
                pltpu.make_async_copy(ones, ones_v, semb).wait()

        @pl.when(c == 0)
        def _():
            run_dir(sa)

        @pl.when(c == 1)
        def _():
            run_dir(sb)

        plsc.subcore_barrier()

        @pl.when(c == 0)
        def _():
            pltpu.sync_copy(acc.at[pl.ds(s * Z_ROWS, Z_ROWS)],
                            oa.at[pl.ds(s * Z_ROWS, Z_ROWS)])

        @pl.when(c == 1)
        def _():
            pltpu.sync_copy(acc.at[pl.ds(s * Z_ROWS, Z_ROWS)],
                            ob.at[pl.ds(s * Z_ROWS, Z_ROWS)])

    return counts


@functools.lru_cache(maxsize=None)
def _make_cls_gather():
    mesh = plsc.VectorSubcoreMesh(core_axis_name="c", subcore_axis_name="s")

    @functools.partial(
        pl.kernel,
        out_type=[jax.ShapeDtypeStruct((P_PAD, H), jnp.float32)] * 2,
        mesh=mesh,
        compiler_params=_sc_params(),
        scratch_types=[
            pltpu.VMEM((KP, CHUNK), jnp.int32),
            pltpu.VMEM((KP, CHUNK, H), jnp.float32),
            pltpu.SemaphoreType.DMA,
        ],
    )
    def cls_gather(ta, tb, aidx, bidx, oa, ob, idx, rows, sem):
        c = lax.axis_index("c")
        s = lax.axis_index("s")

        def run(table, idx_hbm, out):
            def body(g, carry):
                row0 = s * P_TILE + g * KP
                pltpu.sync_copy(idx_hbm.at[pl.ds(row0, KP)], idx)
                cps = [
                    pltpu.async_copy(table.at[idx.at[j]], rows.at[j], sem)
                    for j in range(KP)
                ]
                for cp in cps:
                    cp.wait()
                for j in range(KP):
                    pltpu.sync_copy(
                        rows.at[j], out.at[pl.ds((row0 + j) * CHUNK, CHUNK)])
                return carry
            lax.fori_loop(0, P_TILE // KP, body, 0)

        @pl.when(c == 0)
        def _():
            run(ta, aidx, oa)

        @pl.when(c == 1)
        def _():
            run(tb, bidx, ob)

    return cls_gather


@functools.lru_cache(maxsize=None)
def _neg_tails():
    with jax.ensure_compile_time_eval():
        k1, k2 = jax.random.split(jax.random.key(42))
        na = np.asarray(jax.random.randint(k1, (Q,), 0, N_D, dtype=jnp.int32))
        nb = np.asarray(jax.random.randint(k2, (Q,), 0, N_S, dtype=jnp.int32))
    qpad = P_PAD - 2 * Q
    zpad = np.zeros((qpad,), np.int32)
    return (np.concatenate([na, zpad]), np.concatenate([nb, zpad]))

def kernel(x_pdrugs, x_seffect, node_id_pdrugs, node_id_seffect, edge_index,
           edge_label_index, edge_label, params):
    src = edge_index[0]
    dst = edge_index[1]
    npad = E_PAD - E
    ga = jnp.concatenate([src, jnp.zeros((npad,), jnp.int32)]).reshape(E_ROWS, CHUNK)
    sa = jnp.concatenate([dst, jnp.full((npad,), N_S, jnp.int32)]).reshape(E_ROWS, CHUNK)
    gb = jnp.concatenate([dst, jnp.zeros((npad,), jnp.int32)]).reshape(E_ROWS, CHUNK)
    sb = jnp.concatenate([src, jnp.full((npad,), N_D, jnp.int32)]).reshape(E_ROWS, CHUNK)
    ia = jnp.stack([ga, sa], axis=1)
    ib = jnp.stack([gb, sb], axis=1)

    h_pd, h_se = _proj2(x_pdrugs, params["W_pd"], params["b_pd"],
                        params["emb_pd"], x_seffect, params["W_se"],
                        params["b_se"], params["emb_se"])

    ones = jnp.zeros((CHUNK, CW), jnp.float32).at[:, 0].set(1.0)
    zz16 = jnp.zeros((Z_ROWS, CW), jnp.float32)
    cnt_se, cnt_pd = _make_counts()(sa, sb, ones, zz16)

    z64 = jnp.zeros((Z_ROWS, H), jnp.float32)
    sum_se, sum_pd = _make_aggr()(h_pd, h_se, ia, ib, z64)

    h_se1, h_pd1 = _combine(
        sum_se, cnt_se, h_se, params["W1_ps_l"], params["b1_ps"],
        params["W1_ps_r"], sum_pd, cnt_pd, h_pd, params["W1_sp_l"],
        params["b1_sp"], params["W1_sp_r"], relu=True)

    sum_se2, sum_pd2 = _make_aggr()(h_pd1, h_se1, ia, ib, z64)

    h_se2, h_pd2 = _combine(
        sum_se2, cnt_se, h_se1, params["W2_ps_l"], params["b2_ps"],
        params["W2_ps_r"], sum_pd2, cnt_pd, h_pd1, params["W2_sp_l"],
        params["b2_sp"], params["W2_sp_r"], relu=False)

    na_tail, nb_tail = _neg_tails()
    aidx = jnp.concatenate([edge_label_index[0],
                            jnp.asarray(na_tail)]).reshape(P_ROWS, CHUNK)
    bidx = jnp.concatenate([edge_label_index[1],
                            jnp.asarray(nb_tail)]).reshape(P_ROWS, CHUNK)
    rows_a, rows_b = _make_cls_gather()(h_pd2, h_se2, aidx, bidx)
    pred = _pair_dot(rows_a, rows_b, 2 * Q)
    el = jnp.concatenate([edge_label, jnp.zeros((Q,), jnp.float32)])
    return pred, el
